# baseline (speedup 1.0000x reference)
import functools

import jax
import jax.numpy as jnp
from jax import lax
from jax.experimental import pallas as pl
from jax.experimental.pallas import tpu as pltpu
from jax.experimental.pallas import tpu_sc as plsc

NUM_LATENTS = 1000000
Z_DIM = 64
PAD_DIM = 128
BATCH = 16384

NC, NS = 2, 16
NW = NC * NS
B_PER_W = BATCH // NW
CHUNK = 128
NCHUNK = B_PER_W // CHUNK
L = 16
NG = B_PER_W // L


def _gather_kernel(zp_hbm, idx_hbm, out_hbm, idx_v, pair_v, rows_v, sem):
    wid = lax.axis_index("s") * NC + lax.axis_index("c")
    base = wid * B_PER_W
    pltpu.sync_copy(idx_hbm.at[pl.ds(base, B_PER_W)], idx_v)

    def half(g, carry):
        x = idx_v[pl.ds(g * L, L)]
        pair_v[pl.ds(g * L, L)] = lax.shift_right_logical(x, 1)
        return carry

    lax.fori_loop(0, NG, half, 0)

    for j in range(NCHUNK):
        pltpu.async_copy(
            zp_hbm.at[pair_v.at[pl.ds(j * CHUNK, CHUNK)]],
            rows_v.at[pl.ds(j * CHUNK, CHUNK), :],
            sem,
        )
    pltpu.make_async_copy(zp_hbm.at[pl.ds(0, B_PER_W)], rows_v, sem).wait()

    def select(g, carry):
        k_vec = jax.lax.iota(jnp.int32, L) + g * L
        odd = lax.rem(idx_v[pl.ds(g * L, L)], 2) == 1
        for j0 in range(Z_DIM):
            hi = plsc.load_gather(
                rows_v, [k_vec, jnp.full((L,), Z_DIM + j0, jnp.int32)],
                mask=odd,
            )
            plsc.store_scatter(
                rows_v, [k_vec, jnp.full((L,), j0, jnp.int32)], hi, mask=odd
            )
        return carry

    lax.fori_loop(0, NG, select, 0)
    pltpu.sync_copy(rows_v, out_hbm.at[pl.ds(base, B_PER_W), :])


@jax.jit
def kernel(Z, indices):
    idx = indices.astype(jnp.int32)
    Zp = Z.reshape(NUM_LATENTS // 2, PAD_DIM)
    mesh = plsc.VectorSubcoreMesh(
        core_axis_name="c", subcore_axis_name="s",
        num_cores=NC, num_subcores=NS,
    )
    run = pl.kernel(
        _gather_kernel,
        out_type=jax.ShapeDtypeStruct((BATCH, PAD_DIM), jnp.float32),
        mesh=mesh,
        scratch_types=[
            pltpu.VMEM((B_PER_W,), jnp.int32),
            pltpu.VMEM((B_PER_W,), jnp.int32),
            pltpu.VMEM((B_PER_W, PAD_DIM), jnp.float32),
            pltpu.SemaphoreType.DMA,
        ],
        compiler_params=pltpu.CompilerParams(needs_layout_passes=False),
    )
    return run(Zp, idx)[:, :Z_DIM]


# speedup vs baseline: 1.0591x; 1.0591x over previous
import functools

import jax
import jax.numpy as jnp
from jax import lax
from jax.experimental import pallas as pl
from jax.experimental.pallas import tpu as pltpu
from jax.experimental.pallas import tpu_sc as plsc

NUM_LATENTS = 1000000
Z_DIM = 64
BATCH = 16384

NC, NS = 2, 16
NW = NC * NS
B_PER_W = BATCH // NW
L = 16
WINDOW = 32


def _gather_kernel(zr_hbm, idx_hbm, out_hbm, idx_v, sem):
    wid = lax.axis_index("s") * NC + lax.axis_index("c")
    base = wid * B_PER_W
    pltpu.sync_copy(idx_hbm.at[pl.ds(base, B_PER_W)], idx_v)

    def body(i, carry):
        ivec = plsc.load_gather(idx_v, [jnp.full((L,), 0, jnp.int32) + i])
        c = lax.reduce_max(ivec, (0,))
        pltpu.async_copy(zr_hbm.at[c], out_hbm.at[base + i], sem)

        @pl.when(i >= WINDOW)
        def _drain_one():
            pltpu.make_async_copy(
                zr_hbm.at[0], out_hbm.at[base + i - WINDOW], sem
            ).wait()

        return carry

    lax.fori_loop(0, B_PER_W, body, 0)
    pltpu.make_async_copy(
        zr_hbm.at[pl.ds(0, WINDOW)],
        out_hbm.at[pl.ds(base + B_PER_W - WINDOW, WINDOW)],
        sem,
    ).wait()


@jax.jit
def kernel(Z, indices):
    idx = indices.astype(jnp.int32)
    mesh = plsc.VectorSubcoreMesh(
        core_axis_name="c", subcore_axis_name="s",
        num_cores=NC, num_subcores=NS,
    )
    run = pl.kernel(
        _gather_kernel,
        out_type=jax.ShapeDtypeStruct((BATCH, Z_DIM), jnp.float32),
        mesh=mesh,
        scratch_types=[
            pltpu.VMEM((B_PER_W,), jnp.int32),
            pltpu.SemaphoreType.DMA,
        ],
        compiler_params=pltpu.CompilerParams(needs_layout_passes=False),
    )
    return run(Z, idx)
